# baseline (speedup 1.0000x reference)
import jax
import jax.numpy as jnp
from jax import lax
from jax.experimental import pallas as pl

N, S, K = 4096, 200, 100
R = N * S
R_BLK = 8192


def _body(in_ref, out_ref):
    ids = in_ref[...]
    iota = lax.broadcasted_iota(jnp.int32, (R_BLK, K), 1)
    out_ref[...] = (ids == iota).astype(jnp.float32)


def kernel(inputs):
    flat = inputs.reshape(R, 1)
    out = pl.pallas_call(
        _body,
        grid=(R // R_BLK,),
        in_specs=[pl.BlockSpec((R_BLK, 1), lambda i: (i, 0))],
        out_specs=pl.BlockSpec((R_BLK, K), lambda i: (i, 0)),
        out_shape=jax.ShapeDtypeStruct((R, K), jnp.float32),
    )(flat)
    return out.reshape(N, S, K)
